# baseline (speedup 1.0000x reference)
import functools

import jax
import jax.numpy as jnp
from jax import lax
from jax.experimental import pallas as pl
from jax.experimental.pallas import tpu as pltpu
from jax.experimental.pallas import tpu_sc as plsc

N_REAL = 10000
ROWS = 10240
DUMMY = ROWS - N_REAL
NC = 2
NS = 16
CHUNK = 128
CPT = 80
E_PAD = NC * NS * CPT * CHUNK
RPT = ROWS // NS

D_IN = 128
D_HID = 128
D_CLS = 40
D_CLS_PAD = 64

_MESH = plsc.VectorSubcoreMesh(core_axis_name="c", subcore_axis_name="s")


def _sc_degree(dst_hbm, zeros_hbm, out_hbm, dst_v, ones_v, acc_sh):
    c = lax.axis_index("c")
    s = lax.axis_index("s")
    pltpu.sync_copy(dst_hbm.at[c, s], dst_v)
    sl = pl.ds(s * RPT, RPT)
    pltpu.sync_copy(zeros_hbm.at[sl], acc_sh.at[sl])
    for k in range(CHUNK // 16):
        ones_v[pl.ds(k * 16, 16)] = jnp.ones((16,), jnp.float32)
    plsc.subcore_barrier()

    def body(j, carry):
        pltpu.sync_copy(ones_v, acc_sh.at[dst_v.at[j]], add=True)
        return carry

    lax.fori_loop(0, CPT, body, 0)
    plsc.subcore_barrier()
    pltpu.sync_copy(acc_sh.at[sl], out_hbm.at[c, sl])


_degree_kernel = functools.partial(
    pl.kernel,
    out_type=jax.ShapeDtypeStruct((NC, ROWS), jnp.float32),
    mesh=_MESH,
    scratch_types=[
        pltpu.VMEM((CPT, CHUNK), jnp.int32),
        pltpu.VMEM((CHUNK,), jnp.float32),
        pltpu.VMEM_SHARED((ROWS,), jnp.float32),
    ],
)(_sc_degree)


_IRING = 8


def _sc_agg(R, L, cpt, src_hbm, dst_hbm, y_hbm, zeros_hbm, out_hbm,
            src_v, dst_v, rows_v, acc_sh, *sems):
    isem = sems[0:_IRING]
    gsem = sems[_IRING:_IRING + R]
    ssem = sems[_IRING + R:_IRING + 2 * R]
    c = lax.axis_index("c")
    s = lax.axis_index("s")
    sl = pl.ds(s * RPT, RPT)
    pltpu.sync_copy(zeros_hbm.at[sl], acc_sh.at[sl])
    plsc.subcore_barrier()

    def i_descs(j, k):
        return (pltpu.make_async_copy(src_hbm.at[c, s, j], src_v.at[k], isem[k]),
                pltpu.make_async_copy(dst_hbm.at[c, s, j], dst_v.at[k], isem[k]))

    def g_desc(r, k):
        return pltpu.make_async_copy(
            y_hbm.at[src_v.at[k]], rows_v.at[r], gsem[r])

    def s_desc(r, k):
        return pltpu.make_async_copy(
            rows_v.at[r], acc_sh.at[dst_v.at[k]], ssem[r])

    def start_idx(j, k):
        d1, d2 = i_descs(j, k)
        d1.start()
        d2.start()

    def wait_idx(j, k):
        d1, d2 = i_descs(j, k)
        d1.wait()
        d2.wait()

    def step(k, j0, first, last):
        r = k % R
        g_desc(r, k).wait()
        s_desc(r, k).start(add=True)
        rp = (k - L) % R
        kp = (k - L) % _IRING
        if not (first and k < L):
            s_desc(rp, kp).wait()
            if (not last) or k < L:
                start_idx(j0 + k - L + _IRING, kp)
        if not (last and k >= _IRING - (R - L)):
            kn = (k + R - L) % _IRING
            wait_idx(j0 + k + R - L, kn)
            g_desc(rp, kn).start()

    for k in range(_IRING):
        start_idx(k, k)
    for m in range(R - L):
        wait_idx(m, m)
        g_desc(m % R, m).start()

    for k in range(_IRING):
        step(k, 0, True, cpt == _IRING)

    def body(i, carry):
        j0 = i * _IRING
        for k in range(_IRING):
            step(k, j0, False, False)
        return carry

    lax.fori_loop(1, cpt // _IRING - 1, body, 0)
    for k in range(_IRING):
        step(k, cpt - _IRING, False, True)
    for m in range(L):
        s_desc((_IRING - L + m) % R, _IRING - L + m).wait()
    plsc.subcore_barrier()
    pltpu.sync_copy(acc_sh.at[sl], out_hbm.at[c, sl])


def _make_agg(D, R, L, cpt, chunk, tc_tiling):
    return functools.partial(
        pl.kernel,
        out_type=jax.ShapeDtypeStruct((NC, ROWS, D), jnp.float32),
        mesh=_MESH,
        compiler_params=(None if tc_tiling
                         else pltpu.CompilerParams(use_tc_tiling_on_sc=False)),
        scratch_types=[
            pltpu.VMEM((_IRING, chunk), jnp.int32),
            pltpu.VMEM((_IRING, chunk), jnp.int32),
            pltpu.VMEM((R, chunk, D), jnp.float32),
            pltpu.VMEM_SHARED((ROWS, D), jnp.float32),
        ] + [pltpu.SemaphoreType.DMA] * (_IRING + 2 * R),
    )(functools.partial(_sc_agg, R, L, cpt))


_agg128_kernel = _make_agg(D_HID, 2, 1, CPT, CHUNK, True)
_agg64_kernel = _make_agg(D_CLS_PAD, 4, 2, CPT, CHUNK, False)


_BLK = 512
_GRID = ROWS // _BLK


def _dis_block(deg_ref, i):
    d = deg_ref[0, pl.ds(i * _BLK, _BLK)] + deg_ref[1, pl.ds(i * _BLK, _BLK)]
    return lax.rsqrt(d + 1.0)[:, None]


def _row_mask(i):
    rows = i * _BLK + lax.broadcasted_iota(jnp.int32, (_BLK, 1), 0)
    return (rows < N_REAL).astype(jnp.float32)


def _tc_mm1(x_ref, w_ref, deg_ref, y_ref):
    i = pl.program_id(0)
    dis = _dis_block(deg_ref, i)
    y_ref[...] = jnp.dot(x_ref[...], w_ref[...],
                         preferred_element_type=jnp.float32) * dis


def _tc_hidden(agg_ref, y1_ref, deg_ref, b1_ref, w2_ref, y2_ref):
    i = pl.program_id(0)
    dis = _dis_block(deg_ref, i)
    h = (agg_ref[0] + agg_ref[1] + y1_ref[...]) * dis + b1_ref[...]
    h = jnp.maximum(h, 0.0)
    y2_ref[...] = jnp.dot(h, w2_ref[...],
                          preferred_element_type=jnp.float32) * dis * _row_mask(i)


def _tc_out(agg_ref, y2_ref, deg_ref, b2_ref, o_ref):
    i = pl.program_id(0)
    dis = _dis_block(deg_ref, i)
    z = ((agg_ref[0] + agg_ref[1] + y2_ref[...]) * dis)[:, :D_CLS] + b2_ref[...]
    m = jnp.max(z, axis=1, keepdims=True)
    lse = jnp.log(jnp.sum(jnp.exp(z - m), axis=1, keepdims=True)) + m
    o_ref[...] = z - lse


def _mm1_call(x_pad, W1, degp):
    return pl.pallas_call(
        _tc_mm1,
        grid=(_GRID,),
        in_specs=[
            pl.BlockSpec((_BLK, D_IN), lambda i: (i, 0)),
            pl.BlockSpec((D_IN, D_HID), lambda i: (0, 0)),
            pl.BlockSpec((NC, ROWS), lambda i: (0, 0)),
        ],
        out_specs=pl.BlockSpec((_BLK, D_HID), lambda i: (i, 0)),
        out_shape=jax.ShapeDtypeStruct((ROWS, D_HID), jnp.float32),
    )(x_pad, W1, degp)


def _hidden_call(agg1, y1, degp, b1, W2p):
    return pl.pallas_call(
        _tc_hidden,
        grid=(_GRID,),
        in_specs=[
            pl.BlockSpec((NC, _BLK, D_HID), lambda i: (0, i, 0)),
            pl.BlockSpec((_BLK, D_HID), lambda i: (i, 0)),
            pl.BlockSpec((NC, ROWS), lambda i: (0, 0)),
            pl.BlockSpec((1, D_HID), lambda i: (0, 0)),
            pl.BlockSpec((D_HID, D_CLS_PAD), lambda i: (0, 0)),
        ],
        out_specs=pl.BlockSpec((_BLK, D_CLS_PAD), lambda i: (i, 0)),
        out_shape=jax.ShapeDtypeStruct((ROWS, D_CLS_PAD), jnp.float32),
    )(agg1, y1, degp, b1, W2p)


def _out_call(agg2, y2, degp, b2):
    return pl.pallas_call(
        _tc_out,
        grid=(_GRID,),
        in_specs=[
            pl.BlockSpec((NC, _BLK, D_CLS_PAD), lambda i: (0, i, 0)),
            pl.BlockSpec((_BLK, D_CLS_PAD), lambda i: (i, 0)),
            pl.BlockSpec((NC, ROWS), lambda i: (0, 0)),
            pl.BlockSpec((1, D_CLS), lambda i: (0, 0)),
        ],
        out_specs=pl.BlockSpec((_BLK, D_CLS), lambda i: (i, 0)),
        out_shape=jax.ShapeDtypeStruct((ROWS, D_CLS), jnp.float32),
    )(agg2, y2, degp, b2)


def kernel(x, edge_index, W1, b1, W2, b2):
    ei = edge_index.astype(jnp.int32)
    src = ei[0]
    dst = ei[1]
    pad_n = E_PAD - src.shape[0]
    pad_iota = jnp.arange(pad_n, dtype=jnp.int32)
    pad_rows = N_REAL + pad_iota % DUMMY
    srcp = jnp.concatenate([src, pad_rows]).reshape(NC, NS, CPT, CHUNK)
    dstp = jnp.concatenate([dst, pad_rows]).reshape(NC, NS, CPT, CHUNK)

    zeros_deg = jnp.zeros((ROWS,), jnp.float32)
    zeros128 = jnp.zeros((ROWS, D_HID), jnp.float32)
    zeros64 = jnp.zeros((ROWS, D_CLS_PAD), jnp.float32)
    x_pad = jnp.concatenate([x, jnp.zeros((DUMMY, D_IN), jnp.float32)])
    W2p = jnp.concatenate(
        [W2, jnp.zeros((D_HID, D_CLS_PAD - D_CLS), jnp.float32)], axis=1)

    degp = _degree_kernel(dstp, zeros_deg)
    y1 = _mm1_call(x_pad, W1, degp)
    agg1 = _agg128_kernel(srcp, dstp, y1, zeros128)
    y2 = _hidden_call(agg1, y1, degp, b1.reshape(1, D_HID), W2p)
    agg2 = _agg64_kernel(srcp, dstp, y2, zeros64)
    out = _out_call(agg2, y2, degp, b2.reshape(1, D_CLS))
    return out[:N_REAL]


# speedup vs baseline: 1.1247x; 1.1247x over previous
import functools

import jax
import jax.numpy as jnp
from jax import lax
from jax.experimental import pallas as pl
from jax.experimental.pallas import tpu as pltpu
from jax.experimental.pallas import tpu_sc as plsc

N_REAL = 10000
ROWS = 10240
DUMMY = ROWS - N_REAL
NC = 2
NS = 16
CHUNK = 128
CPT = 80
E_PAD = NC * NS * CPT * CHUNK
RPT = ROWS // NS

D_IN = 128
D_HID = 128
D_CLS = 40
D_CLS_PAD = 64

_MESH = plsc.VectorSubcoreMesh(core_axis_name="c", subcore_axis_name="s")


def _sc_degree(dst_hbm, zeros_hbm, out_hbm, dst_v, ones_v, acc_sh):
    c = lax.axis_index("c")
    s = lax.axis_index("s")
    pltpu.sync_copy(dst_hbm.at[c, s], dst_v)
    sl = pl.ds(s * RPT, RPT)
    pltpu.sync_copy(zeros_hbm.at[sl], acc_sh.at[sl])
    for k in range(CHUNK // 16):
        ones_v[pl.ds(k * 16, 16)] = jnp.ones((16,), jnp.float32)
    plsc.subcore_barrier()

    def body(j, carry):
        pltpu.sync_copy(ones_v, acc_sh.at[dst_v.at[j]], add=True)
        return carry

    lax.fori_loop(0, CPT, body, 0)
    plsc.subcore_barrier()
    pltpu.sync_copy(acc_sh.at[sl], out_hbm.at[c, sl])


_degree_kernel = functools.partial(
    pl.kernel,
    out_type=jax.ShapeDtypeStruct((NC, ROWS), jnp.float32),
    mesh=_MESH,
    scratch_types=[
        pltpu.VMEM((CPT, CHUNK), jnp.int32),
        pltpu.VMEM((CHUNK,), jnp.float32),
        pltpu.VMEM_SHARED((ROWS,), jnp.float32),
    ],
)(_sc_degree)


_IRING = 8


def _sc_agg(R, L, cpt, src_hbm, dst_hbm, y_hbm, zeros_hbm, out_hbm,
            src_v, dst_v, rows_v, acc_sh, *sems):
    isem = sems[0:_IRING]
    gsem = sems[_IRING:_IRING + R]
    ssem = sems[_IRING + R:_IRING + 2 * R]
    c = lax.axis_index("c")
    s = lax.axis_index("s")
    sl = pl.ds(s * RPT, RPT)

    @pl.when(c == 0)
    def _():
        pltpu.sync_copy(y_hbm.at[sl], acc_sh.at[sl])

    @pl.when(c != 0)
    def _():
        pltpu.sync_copy(zeros_hbm.at[sl], acc_sh.at[sl])

    plsc.subcore_barrier()

    def i_descs(j, k):
        return (pltpu.make_async_copy(src_hbm.at[c, s, j], src_v.at[k], isem[k]),
                pltpu.make_async_copy(dst_hbm.at[c, s, j], dst_v.at[k], isem[k]))

    def g_desc(r, k):
        return pltpu.make_async_copy(
            y_hbm.at[src_v.at[k]], rows_v.at[r], gsem[r])

    def s_desc(r, k):
        return pltpu.make_async_copy(
            rows_v.at[r], acc_sh.at[dst_v.at[k]], ssem[r])

    def start_idx(j, k):
        d1, d2 = i_descs(j, k)
        d1.start()
        d2.start()

    def wait_idx(j, k):
        d1, d2 = i_descs(j, k)
        d1.wait()
        d2.wait()

    def step(k, j0, first, last):
        r = k % R
        g_desc(r, k).wait()
        s_desc(r, k).start(add=True)
        rp = (k - L) % R
        kp = (k - L) % _IRING
        if not (first and k < L):
            s_desc(rp, kp).wait()
            if (not last) or k < L:
                start_idx(j0 + k - L + _IRING, kp)
        if not (last and k >= _IRING - (R - L)):
            kn = (k + R - L) % _IRING
            wait_idx(j0 + k + R - L, kn)
            g_desc(rp, kn).start()

    for k in range(_IRING):
        start_idx(k, k)
    for m in range(R - L):
        wait_idx(m, m)
        g_desc(m % R, m).start()

    for k in range(_IRING):
        step(k, 0, True, cpt == _IRING)

    def body(i, carry):
        j0 = i * _IRING
        for k in range(_IRING):
            step(k, j0, False, False)
        return carry

    lax.fori_loop(1, cpt // _IRING - 1, body, 0)
    for k in range(_IRING):
        step(k, cpt - _IRING, False, True)
    for m in range(L):
        s_desc((_IRING - L + m) % R, _IRING - L + m).wait()
    plsc.subcore_barrier()
    pltpu.sync_copy(acc_sh.at[sl], out_hbm.at[c, sl])


def _make_agg(D, R, L, cpt, chunk, tc_tiling):
    return functools.partial(
        pl.kernel,
        out_type=jax.ShapeDtypeStruct((NC, ROWS, D), jnp.float32),
        mesh=_MESH,
        compiler_params=(None if tc_tiling
                         else pltpu.CompilerParams(use_tc_tiling_on_sc=False)),
        scratch_types=[
            pltpu.VMEM((_IRING, chunk), jnp.int32),
            pltpu.VMEM((_IRING, chunk), jnp.int32),
            pltpu.VMEM((R, chunk, D), jnp.float32),
            pltpu.VMEM_SHARED((ROWS, D), jnp.float32),
        ] + [pltpu.SemaphoreType.DMA] * (_IRING + 2 * R),
    )(functools.partial(_sc_agg, R, L, cpt))


_agg128_kernel = _make_agg(D_HID, 2, 0, CPT, CHUNK, True)
_agg64_kernel = _make_agg(D_CLS_PAD, 4, 0, CPT, CHUNK, False)


_BLK = 512
_GRID = ROWS // _BLK


def _dis_block(deg_ref, i):
    d = deg_ref[0, pl.ds(i * _BLK, _BLK)] + deg_ref[1, pl.ds(i * _BLK, _BLK)]
    return lax.rsqrt(d + 1.0)[:, None]


def _row_mask(i):
    rows = i * _BLK + lax.broadcasted_iota(jnp.int32, (_BLK, 1), 0)
    return (rows < N_REAL).astype(jnp.float32)


def _tc_mm0(x_ref, w_ref, y_ref):
    y_ref[...] = jnp.dot(x_ref[...], w_ref[...],
                         preferred_element_type=jnp.float32)


def _tc_scale(xw_ref, deg_ref, y_ref):
    i = pl.program_id(0)
    y_ref[...] = xw_ref[...] * _dis_block(deg_ref, i)


def _tc_hidden(agg_ref, deg_ref, b1_ref, w2_ref, y2_ref):
    i = pl.program_id(0)
    dis = _dis_block(deg_ref, i)
    h = (agg_ref[0] + agg_ref[1]) * dis + b1_ref[...]
    h = jnp.maximum(h, 0.0)
    y2_ref[...] = jnp.dot(h, w2_ref[...],
                          preferred_element_type=jnp.float32) * dis * _row_mask(i)


def _tc_out(agg_ref, deg_ref, b2_ref, o_ref):
    i = pl.program_id(0)
    dis = _dis_block(deg_ref, i)
    z = ((agg_ref[0] + agg_ref[1]) * dis)[:, :D_CLS] + b2_ref[...]
    m = jnp.max(z, axis=1, keepdims=True)
    lse = jnp.log(jnp.sum(jnp.exp(z - m), axis=1, keepdims=True)) + m
    o_ref[...] = z - lse


def _mm0_call(x_pad, W1):
    return pl.pallas_call(
        _tc_mm0,
        grid=(_GRID,),
        in_specs=[
            pl.BlockSpec((_BLK, D_IN), lambda i: (i, 0)),
            pl.BlockSpec((D_IN, D_HID), lambda i: (0, 0)),
        ],
        out_specs=pl.BlockSpec((_BLK, D_HID), lambda i: (i, 0)),
        out_shape=jax.ShapeDtypeStruct((ROWS, D_HID), jnp.float32),
    )(x_pad, W1)


def _scale_call(xw, degp):
    return pl.pallas_call(
        _tc_scale,
        grid=(_GRID,),
        in_specs=[
            pl.BlockSpec((_BLK, D_HID), lambda i: (i, 0)),
            pl.BlockSpec((NC, ROWS), lambda i: (0, 0)),
        ],
        out_specs=pl.BlockSpec((_BLK, D_HID), lambda i: (i, 0)),
        out_shape=jax.ShapeDtypeStruct((ROWS, D_HID), jnp.float32),
    )(xw, degp)


def _hidden_call(agg1, degp, b1, W2p):
    return pl.pallas_call(
        _tc_hidden,
        grid=(_GRID,),
        in_specs=[
            pl.BlockSpec((NC, _BLK, D_HID), lambda i: (0, i, 0)),
            pl.BlockSpec((NC, ROWS), lambda i: (0, 0)),
            pl.BlockSpec((1, D_HID), lambda i: (0, 0)),
            pl.BlockSpec((D_HID, D_CLS_PAD), lambda i: (0, 0)),
        ],
        out_specs=pl.BlockSpec((_BLK, D_CLS_PAD), lambda i: (i, 0)),
        out_shape=jax.ShapeDtypeStruct((ROWS, D_CLS_PAD), jnp.float32),
    )(agg1, degp, b1, W2p)


def _out_call(agg2, degp, b2):
    return pl.pallas_call(
        _tc_out,
        grid=(_GRID,),
        in_specs=[
            pl.BlockSpec((NC, _BLK, D_CLS_PAD), lambda i: (0, i, 0)),
            pl.BlockSpec((NC, ROWS), lambda i: (0, 0)),
            pl.BlockSpec((1, D_CLS), lambda i: (0, 0)),
        ],
        out_specs=pl.BlockSpec((_BLK, D_CLS), lambda i: (i, 0)),
        out_shape=jax.ShapeDtypeStruct((ROWS, D_CLS), jnp.float32),
    )(agg2, degp, b2)


def kernel(x, edge_index, W1, b1, W2, b2):
    ei = edge_index.astype(jnp.int32)
    src = ei[0]
    dst = ei[1]
    pad_n = E_PAD - src.shape[0]
    pad_iota = jnp.arange(pad_n, dtype=jnp.int32)
    pad_rows = N_REAL + pad_iota % DUMMY
    srcp = jnp.concatenate([src, pad_rows]).reshape(NC, NS, CPT, CHUNK)
    dstp = jnp.concatenate([dst, pad_rows]).reshape(NC, NS, CPT, CHUNK)

    zeros_deg = jnp.zeros((ROWS,), jnp.float32)
    zeros128 = jnp.zeros((ROWS, D_HID), jnp.float32)
    zeros64 = jnp.zeros((ROWS, D_CLS_PAD), jnp.float32)
    x_pad = jnp.concatenate([x, jnp.zeros((DUMMY, D_IN), jnp.float32)])
    W2p = jnp.concatenate(
        [W2, jnp.zeros((D_HID, D_CLS_PAD - D_CLS), jnp.float32)], axis=1)

    degp = _degree_kernel(dstp, zeros_deg)
    xw = _mm0_call(x_pad, W1)
    y1 = _scale_call(xw, degp)
    agg1 = _agg128_kernel(srcp, dstp, y1, zeros128)
    y2 = _hidden_call(agg1, degp, b1.reshape(1, D_HID), W2p)
    agg2 = _agg64_kernel(srcp, dstp, y2, zeros64)
    out = _out_call(agg2, degp, b2.reshape(1, D_CLS))
    return out[:N_REAL]


# speedup vs baseline: 1.1415x; 1.0149x over previous
import functools

import jax
import jax.numpy as jnp
from jax import lax
from jax.experimental import pallas as pl
from jax.experimental.pallas import tpu as pltpu
from jax.experimental.pallas import tpu_sc as plsc

N_REAL = 10000
ROWS = 10240
DUMMY = ROWS - N_REAL
NC = 2
NS = 16
CHUNK = 128
CPT = 80
E_PAD = NC * NS * CPT * CHUNK
RPT = ROWS // NS

D_IN = 128
D_HID = 128
D_CLS = 40
D_CLS_PAD = 64

_MESH = plsc.VectorSubcoreMesh(core_axis_name="c", subcore_axis_name="s")


def _sc_degree(dst_hbm, zeros_hbm, out_hbm, dst_v, ones_v, acc_sh):
    c = lax.axis_index("c")
    s = lax.axis_index("s")
    pltpu.sync_copy(dst_hbm.at[c, s], dst_v)
    sl = pl.ds(s * RPT, RPT)
    pltpu.sync_copy(zeros_hbm, acc_sh.at[sl])
    for k in range(CHUNK // 16):
        ones_v[pl.ds(k * 16, 16)] = jnp.ones((16,), jnp.float32)
    plsc.subcore_barrier()

    def body(j, carry):
        pltpu.sync_copy(ones_v, acc_sh.at[dst_v.at[j]], add=True)
        return carry

    lax.fori_loop(0, CPT, body, 0)
    plsc.subcore_barrier()
    pltpu.sync_copy(acc_sh.at[sl], out_hbm.at[c, sl])


_degree_kernel = functools.partial(
    pl.kernel,
    out_type=jax.ShapeDtypeStruct((NC, ROWS), jnp.float32),
    mesh=_MESH,
    scratch_types=[
        pltpu.VMEM((CPT, CHUNK), jnp.int32),
        pltpu.VMEM((CHUNK,), jnp.float32),
        pltpu.VMEM_SHARED((ROWS,), jnp.float32),
    ],
)(_sc_degree)


_IRING = 8


def _sc_agg(R, L, cpt, src_hbm, dst_hbm, y_hbm, zeros_hbm, out_hbm,
            src_v, dst_v, rows_v, acc_sh, *sems):
    isem = sems[0:_IRING]
    gsem = sems[_IRING:_IRING + R]
    ssem = sems[_IRING + R:_IRING + 2 * R]
    c = lax.axis_index("c")
    s = lax.axis_index("s")
    sl = pl.ds(s * RPT, RPT)

    @pl.when(c == 0)
    def _():
        pltpu.sync_copy(y_hbm.at[sl], acc_sh.at[sl])

    @pl.when(c != 0)
    def _():
        pltpu.sync_copy(zeros_hbm, acc_sh.at[sl])

    plsc.subcore_barrier()

    def i_descs(j, k):
        return (pltpu.make_async_copy(src_hbm.at[c, s, j], src_v.at[k], isem[k]),
                pltpu.make_async_copy(dst_hbm.at[c, s, j], dst_v.at[k], isem[k]))

    def g_desc(r, k):
        return pltpu.make_async_copy(
            y_hbm.at[src_v.at[k]], rows_v.at[r], gsem[r])

    def s_desc(r, k):
        return pltpu.make_async_copy(
            rows_v.at[r], acc_sh.at[dst_v.at[k]], ssem[r])

    def start_idx(j, k):
        d1, d2 = i_descs(j, k)
        d1.start()
        d2.start()

    def wait_idx(j, k):
        d1, d2 = i_descs(j, k)
        d1.wait()
        d2.wait()

    def step(k, j0, first, last):
        r = k % R
        g_desc(r, k).wait()
        s_desc(r, k).start(add=True)
        rp = (k - L) % R
        kp = (k - L) % _IRING
        if not (first and k < L):
            s_desc(rp, kp).wait()
            if (not last) or k < L:
                start_idx(j0 + k - L + _IRING, kp)
        if not (last and k >= _IRING - (R - L)):
            kn = (k + R - L) % _IRING
            wait_idx(j0 + k + R - L, kn)
            g_desc(rp, kn).start()

    for k in range(_IRING):
        start_idx(k, k)
    for m in range(R - L):
        wait_idx(m, m)
        g_desc(m % R, m).start()

    for k in range(_IRING):
        step(k, 0, True, cpt == _IRING)

    def body(i, carry):
        j0 = i * _IRING
        for k in range(_IRING):
            step(k, j0, False, False)
        return carry

    lax.fori_loop(1, cpt // _IRING - 1, body, 0)
    for k in range(_IRING):
        step(k, cpt - _IRING, False, True)
    for m in range(L):
        s_desc((_IRING - L + m) % R, _IRING - L + m).wait()
    plsc.subcore_barrier()
    pltpu.sync_copy(acc_sh.at[sl], out_hbm.at[c, sl])


def _make_agg(D, R, L, cpt, chunk, tc_tiling):
    return functools.partial(
        pl.kernel,
        out_type=jax.ShapeDtypeStruct((NC, ROWS, D), jnp.float32),
        mesh=_MESH,
        compiler_params=(None if tc_tiling
                         else pltpu.CompilerParams(use_tc_tiling_on_sc=False)),
        scratch_types=[
            pltpu.VMEM((_IRING, chunk), jnp.int32),
            pltpu.VMEM((_IRING, chunk), jnp.int32),
            pltpu.VMEM((R, chunk, D), jnp.float32),
            pltpu.VMEM_SHARED((ROWS, D), jnp.float32),
        ] + [pltpu.SemaphoreType.DMA] * (_IRING + 2 * R),
    )(functools.partial(_sc_agg, R, L, cpt))


_agg128_kernel = _make_agg(D_HID, 2, 0, CPT, CHUNK, True)
_agg64_kernel = _make_agg(D_CLS_PAD, 4, 0, CPT, CHUNK, False)


_BLK = 1024
_GRID = ROWS // _BLK
_OBLK = 400
_OGRID = N_REAL // _OBLK


def _dis(degt_block):
    return lax.rsqrt(degt_block[:, 0] + degt_block[:, 1] + 1.0)[:, None]


def _tc_mm1(x_ref, w_ref, degt_ref, y_ref):
    y_ref[...] = jnp.dot(x_ref[...], w_ref[...],
                         preferred_element_type=jnp.float32) * _dis(degt_ref[...])


def _tc_hidden(agg_ref, degt_ref, b1_ref, w2_ref, y2_ref):
    i = pl.program_id(0)
    dis = _dis(degt_ref[...])
    h = (agg_ref[0] + agg_ref[1]) * dis + b1_ref[...]
    h = jnp.maximum(h, 0.0)
    rows = i * _BLK + lax.broadcasted_iota(jnp.int32, (_BLK, 1), 0)
    mask = (rows < N_REAL).astype(jnp.float32)
    y2_ref[...] = jnp.dot(h, w2_ref[...],
                          preferred_element_type=jnp.float32) * dis * mask


def _tc_out(agg_ref, degt_ref, b2_ref, o_ref):
    dis = _dis(degt_ref[...])
    z = ((agg_ref[0] + agg_ref[1]) * dis)[:, :D_CLS] + b2_ref[...]
    m = jnp.max(z, axis=1, keepdims=True)
    lse = jnp.log(jnp.sum(jnp.exp(z - m), axis=1, keepdims=True)) + m
    o_ref[...] = z - lse


def _mm1_call(x_pad, W1, degt):
    return pl.pallas_call(
        _tc_mm1,
        grid=(_GRID,),
        in_specs=[
            pl.BlockSpec((_BLK, D_IN), lambda i: (i, 0)),
            pl.BlockSpec((D_IN, D_HID), lambda i: (0, 0)),
            pl.BlockSpec((_BLK, NC), lambda i: (i, 0)),
        ],
        out_specs=pl.BlockSpec((_BLK, D_HID), lambda i: (i, 0)),
        out_shape=jax.ShapeDtypeStruct((ROWS, D_HID), jnp.float32),
    )(x_pad, W1, degt)


def _hidden_call(agg1, degt, b1, W2p):
    return pl.pallas_call(
        _tc_hidden,
        grid=(_GRID,),
        in_specs=[
            pl.BlockSpec((NC, _BLK, D_HID), lambda i: (0, i, 0)),
            pl.BlockSpec((_BLK, NC), lambda i: (i, 0)),
            pl.BlockSpec((1, D_HID), lambda i: (0, 0)),
            pl.BlockSpec((D_HID, D_CLS_PAD), lambda i: (0, 0)),
        ],
        out_specs=pl.BlockSpec((_BLK, D_CLS_PAD), lambda i: (i, 0)),
        out_shape=jax.ShapeDtypeStruct((ROWS, D_CLS_PAD), jnp.float32),
    )(agg1, degt, b1, W2p)


def _out_call(agg2, degt, b2):
    return pl.pallas_call(
        _tc_out,
        grid=(_OGRID,),
        in_specs=[
            pl.BlockSpec((NC, _OBLK, D_CLS_PAD), lambda i: (0, i, 0)),
            pl.BlockSpec((_OBLK, NC), lambda i: (i, 0)),
            pl.BlockSpec((1, D_CLS), lambda i: (0, 0)),
        ],
        out_specs=pl.BlockSpec((_OBLK, D_CLS), lambda i: (i, 0)),
        out_shape=jax.ShapeDtypeStruct((N_REAL, D_CLS), jnp.float32),
    )(agg2, degt, b2)


def kernel(x, edge_index, W1, b1, W2, b2):
    ei = edge_index.astype(jnp.int32)
    src = ei[0]
    dst = ei[1]
    pad_n = E_PAD - src.shape[0]
    pad_iota = jnp.arange(pad_n, dtype=jnp.int32)
    pad_rows = N_REAL + pad_iota % DUMMY
    srcp = jnp.concatenate([src, pad_rows]).reshape(NC, NS, CPT, CHUNK)
    dstp = jnp.concatenate([dst, pad_rows]).reshape(NC, NS, CPT, CHUNK)

    zeros_deg = jnp.zeros((RPT,), jnp.float32)
    zeros128 = jnp.zeros((RPT, D_HID), jnp.float32)
    zeros64 = jnp.zeros((RPT, D_CLS_PAD), jnp.float32)
    x_pad = jnp.concatenate([x, jnp.zeros((DUMMY, D_IN), jnp.float32)])
    W2p = jnp.concatenate(
        [W2, jnp.zeros((D_HID, D_CLS_PAD - D_CLS), jnp.float32)], axis=1)

    degp = _degree_kernel(dstp, zeros_deg)
    degt = degp.T
    y1 = _mm1_call(x_pad, W1, degt)
    agg1 = _agg128_kernel(srcp, dstp, y1, zeros128)
    y2 = _hidden_call(agg1, degt, b1.reshape(1, D_HID), W2p)
    agg2 = _agg64_kernel(srcp, dstp, y2, zeros64)
    return _out_call(agg2, degt, b2.reshape(1, D_CLS))


# speedup vs baseline: 1.1607x; 1.0169x over previous
import functools

import jax
import jax.numpy as jnp
from jax import lax
from jax.experimental import pallas as pl
from jax.experimental.pallas import tpu as pltpu
from jax.experimental.pallas import tpu_sc as plsc

N_REAL = 10000
ROWS = 10240
DUMMY = ROWS - N_REAL
NC = 2
NS = 16
CHUNK = 128
CPT = 80
E_PAD = NC * NS * CPT * CHUNK
RPT = ROWS // NS

D_IN = 128
D_HID = 128
D_CLS = 40
D_CLS_PAD = 64

_MESH = plsc.VectorSubcoreMesh(core_axis_name="c", subcore_axis_name="s")


def _sc_degree(dst_hbm, zeros_hbm, out_hbm, dst_v, ones_v, acc_sh):
    c = lax.axis_index("c")
    s = lax.axis_index("s")
    pltpu.sync_copy(dst_hbm.at[c, s], dst_v)
    sl = pl.ds(s * RPT, RPT)
    pltpu.sync_copy(zeros_hbm, acc_sh.at[sl])
    for k in range(CHUNK // 16):
        ones_v[pl.ds(k * 16, 16)] = jnp.ones((16,), jnp.float32)
    plsc.subcore_barrier()

    def body(j, carry):
        pltpu.sync_copy(ones_v, acc_sh.at[dst_v.at[j]], add=True)
        return carry

    lax.fori_loop(0, CPT, body, 0)
    plsc.subcore_barrier()
    pltpu.sync_copy(acc_sh.at[sl], out_hbm.at[c, sl])


_degree_kernel = functools.partial(
    pl.kernel,
    out_type=jax.ShapeDtypeStruct((NC, ROWS), jnp.float32),
    mesh=_MESH,
    scratch_types=[
        pltpu.VMEM((CPT, CHUNK), jnp.int32),
        pltpu.VMEM((CHUNK,), jnp.float32),
        pltpu.VMEM_SHARED((ROWS,), jnp.float32),
    ],
)(_sc_degree)


_IRING = 8


def _sc_agg(R, L, cpt, src_hbm, dst_hbm, y_hbm, zeros_hbm, out_hbm,
            src_v, dst_v, rows_v, acc_sh, *sems):
    isem = sems[0:_IRING]
    gsem = sems[_IRING:_IRING + R]
    ssem = sems[_IRING + R:_IRING + 2 * R]
    c = lax.axis_index("c")
    s = lax.axis_index("s")
    sl = pl.ds(s * RPT, RPT)

    @pl.when(c == 0)
    def _():
        pltpu.sync_copy(y_hbm.at[sl], acc_sh.at[sl])

    @pl.when(c != 0)
    def _():
        pltpu.sync_copy(zeros_hbm, acc_sh.at[sl])

    plsc.subcore_barrier()

    def i_descs(j, k):
        return (pltpu.make_async_copy(src_hbm.at[c, s, j], src_v.at[k], isem[k]),
                pltpu.make_async_copy(dst_hbm.at[c, s, j], dst_v.at[k], isem[k]))

    def g_desc(r, k):
        return pltpu.make_async_copy(
            y_hbm.at[src_v.at[k]], rows_v.at[r], gsem[r])

    def s_desc(r, k):
        return pltpu.make_async_copy(
            rows_v.at[r], acc_sh.at[dst_v.at[k]], ssem[r])

    def start_idx(j, k):
        d1, d2 = i_descs(j, k)
        d1.start()
        d2.start()

    def wait_idx(j, k):
        d1, d2 = i_descs(j, k)
        d1.wait()
        d2.wait()

    def step(k, j0, first, last):
        r = k % R
        g_desc(r, k).wait()
        s_desc(r, k).start(add=True)
        rp = (k - L) % R
        kp = (k - L) % _IRING
        if not (first and k < L):
            s_desc(rp, kp).wait()
            if (not last) or k < L:
                start_idx(j0 + k - L + _IRING, kp)
        if not (last and k >= _IRING - (R - L)):
            kn = (k + R - L) % _IRING
            wait_idx(j0 + k + R - L, kn)
            g_desc(rp, kn).start()

    for k in range(_IRING):
        start_idx(k, k)
    for m in range(R - L):
        wait_idx(m, m)
        g_desc(m % R, m).start()

    for k in range(_IRING):
        step(k, 0, True, cpt == _IRING)

    def body(i, carry):
        j0 = i * _IRING
        for k in range(_IRING):
            step(k, j0, False, False)
        return carry

    lax.fori_loop(1, cpt // _IRING - 1, body, 0)
    for k in range(_IRING):
        step(k, cpt - _IRING, False, True)
    for m in range(L):
        s_desc((_IRING - L + m) % R, _IRING - L + m).wait()
    plsc.subcore_barrier()
    pltpu.sync_copy(acc_sh.at[sl], out_hbm.at[c, sl])


def _make_agg(D, R, L, cpt, chunk, tc_tiling):
    return functools.partial(
        pl.kernel,
        out_type=jax.ShapeDtypeStruct((NC, ROWS, D), jnp.float32),
        mesh=_MESH,
        compiler_params=(None if tc_tiling
                         else pltpu.CompilerParams(use_tc_tiling_on_sc=False)),
        scratch_types=[
            pltpu.VMEM((_IRING, chunk), jnp.int32),
            pltpu.VMEM((_IRING, chunk), jnp.int32),
            pltpu.VMEM((R, chunk, D), jnp.float32),
            pltpu.VMEM_SHARED((ROWS, D), jnp.float32),
        ] + [pltpu.SemaphoreType.DMA] * (_IRING + 2 * R),
    )(functools.partial(_sc_agg, R, L, cpt))


_agg128_kernel = _make_agg(D_HID, 2, 0, CPT, CHUNK, True)
_agg64_kernel = _make_agg(D_CLS_PAD, 4, 0, CPT, CHUNK, False)


_BLK = 2048
_GRID = ROWS // _BLK
_OBLK = 400
_OGRID = N_REAL // _OBLK


def _dis(degt_block):
    return lax.rsqrt(degt_block[:, 0] + degt_block[:, 1] + 1.0)[:, None]


def _tc_mm1(x_ref, w_ref, degt_ref, y_ref):
    y_ref[...] = jnp.dot(x_ref[...], w_ref[...],
                         preferred_element_type=jnp.float32) * _dis(degt_ref[...])


def _tc_hidden(agg_ref, degt_ref, b1_ref, w2_ref, y2_ref):
    i = pl.program_id(0)
    dis = _dis(degt_ref[...])
    h = (agg_ref[0] + agg_ref[1]) * dis + b1_ref[...]
    h = jnp.maximum(h, 0.0)
    rows = i * _BLK + lax.broadcasted_iota(jnp.int32, (_BLK, 1), 0)
    mask = (rows < N_REAL).astype(jnp.float32)
    y2_ref[...] = jnp.dot(h, w2_ref[...],
                          preferred_element_type=jnp.float32) * dis * mask


def _tc_out(agg_ref, degt_ref, b2_ref, o_ref):
    dis = _dis(degt_ref[...])
    z = ((agg_ref[0] + agg_ref[1]) * dis)[:, :D_CLS] + b2_ref[...]
    m = jnp.max(z, axis=1, keepdims=True)
    lse = jnp.log(jnp.sum(jnp.exp(z - m), axis=1, keepdims=True)) + m
    o_ref[...] = z - lse


def _mm1_call(x_pad, W1, degt):
    return pl.pallas_call(
        _tc_mm1,
        grid=(_GRID,),
        in_specs=[
            pl.BlockSpec((_BLK, D_IN), lambda i: (i, 0)),
            pl.BlockSpec((D_IN, D_HID), lambda i: (0, 0)),
            pl.BlockSpec((_BLK, NC), lambda i: (i, 0)),
        ],
        out_specs=pl.BlockSpec((_BLK, D_HID), lambda i: (i, 0)),
        out_shape=jax.ShapeDtypeStruct((ROWS, D_HID), jnp.float32),
    )(x_pad, W1, degt)


def _hidden_call(agg1, degt, b1, W2p):
    return pl.pallas_call(
        _tc_hidden,
        grid=(_GRID,),
        in_specs=[
            pl.BlockSpec((NC, _BLK, D_HID), lambda i: (0, i, 0)),
            pl.BlockSpec((_BLK, NC), lambda i: (i, 0)),
            pl.BlockSpec((1, D_HID), lambda i: (0, 0)),
            pl.BlockSpec((D_HID, D_CLS_PAD), lambda i: (0, 0)),
        ],
        out_specs=pl.BlockSpec((_BLK, D_CLS_PAD), lambda i: (i, 0)),
        out_shape=jax.ShapeDtypeStruct((ROWS, D_CLS_PAD), jnp.float32),
    )(agg1, degt, b1, W2p)


def _out_call(agg2, degt, b2):
    return pl.pallas_call(
        _tc_out,
        grid=(_OGRID,),
        in_specs=[
            pl.BlockSpec((NC, _OBLK, D_CLS_PAD), lambda i: (0, i, 0)),
            pl.BlockSpec((_OBLK, NC), lambda i: (i, 0)),
            pl.BlockSpec((1, D_CLS), lambda i: (0, 0)),
        ],
        out_specs=pl.BlockSpec((_OBLK, D_CLS), lambda i: (i, 0)),
        out_shape=jax.ShapeDtypeStruct((N_REAL, D_CLS), jnp.float32),
    )(agg2, degt, b2)


def kernel(x, edge_index, W1, b1, W2, b2):
    ei = edge_index.astype(jnp.int32)
    src = ei[0]
    dst = ei[1]
    pad_n = E_PAD - src.shape[0]
    pad_iota = jnp.arange(pad_n, dtype=jnp.int32)
    pad_rows = N_REAL + pad_iota % DUMMY
    srcp = jnp.concatenate([src, pad_rows]).reshape(NC, NS, CPT, CHUNK)
    dstp = jnp.concatenate([dst, pad_rows]).reshape(NC, NS, CPT, CHUNK)

    zeros_deg = jnp.zeros((RPT,), jnp.float32)
    zeros128 = jnp.zeros((RPT, D_HID), jnp.float32)
    zeros64 = jnp.zeros((RPT, D_CLS_PAD), jnp.float32)
    x_pad = jnp.concatenate([x, jnp.zeros((DUMMY, D_IN), jnp.float32)])
    W2p = jnp.concatenate(
        [W2, jnp.zeros((D_HID, D_CLS_PAD - D_CLS), jnp.float32)], axis=1)

    degp = _degree_kernel(dstp, zeros_deg)
    degt = degp.T
    y1 = _mm1_call(x_pad, W1, degt)
    agg1 = _agg128_kernel(srcp, dstp, y1, zeros128)
    y2 = _hidden_call(agg1, degt, b1.reshape(1, D_HID), W2p)
    agg2 = _agg64_kernel(srcp, dstp, y2, zeros64)
    return _out_call(agg2, degt, b2.reshape(1, D_CLS))


# speedup vs baseline: 1.1812x; 1.0177x over previous
import functools

import jax
import jax.numpy as jnp
from jax import lax
from jax.experimental import pallas as pl
from jax.experimental.pallas import tpu as pltpu
from jax.experimental.pallas import tpu_sc as plsc

N_REAL = 10000
ROWS = 10240
DUMMY = ROWS - N_REAL
NC = 2
NS = 16
CHUNK = 128
CPT = 80
E_PAD = NC * NS * CPT * CHUNK
RPT = ROWS // NS

D_IN = 128
D_HID = 128
D_CLS = 40
D_CLS_PAD = 64

_MESH = plsc.VectorSubcoreMesh(core_axis_name="c", subcore_axis_name="s")


def _sc_degree(dst_hbm, zeros_hbm, out_hbm, dst_v, ones_v, acc_sh, sem):
    c = lax.axis_index("c")
    s = lax.axis_index("s")
    pltpu.sync_copy(dst_hbm.at[c, s], dst_v)
    sl = pl.ds(s * RPT, RPT)
    pltpu.sync_copy(zeros_hbm, acc_sh.at[sl])
    for k in range(CHUNK // 16):
        ones_v[pl.ds(k * 16, 16)] = jnp.ones((16,), jnp.float32)
    plsc.subcore_barrier()

    def desc(j):
        return pltpu.make_async_copy(ones_v, acc_sh.at[dst_v.at[j]], sem)

    def fire(j, carry):
        desc(j).start(add=True)
        return carry

    def drain(j, carry):
        desc(j).wait()
        return carry

    lax.fori_loop(0, CPT, fire, 0)
    lax.fori_loop(0, CPT, drain, 0)
    plsc.subcore_barrier()
    pltpu.sync_copy(acc_sh.at[sl], out_hbm.at[c, sl])


_degree_kernel = functools.partial(
    pl.kernel,
    out_type=jax.ShapeDtypeStruct((NC, ROWS), jnp.float32),
    mesh=_MESH,
    scratch_types=[
        pltpu.VMEM((CPT, CHUNK), jnp.int32),
        pltpu.VMEM((CHUNK,), jnp.float32),
        pltpu.VMEM_SHARED((ROWS,), jnp.float32),
        pltpu.SemaphoreType.DMA,
    ],
)(_sc_degree)


_IRING = 8


def _sc_agg(R, L, cpt, src_hbm, dst_hbm, y_hbm, zeros_hbm, out_hbm,
            src_v, dst_v, rows_v, acc_sh, *sems):
    isem = sems[0:_IRING]
    gsem = sems[_IRING:_IRING + R]
    ssem = sems[_IRING + R:_IRING + 2 * R]
    c = lax.axis_index("c")
    s = lax.axis_index("s")
    sl = pl.ds(s * RPT, RPT)

    @pl.when(c == 0)
    def _():
        pltpu.sync_copy(y_hbm.at[sl], acc_sh.at[sl])

    @pl.when(c != 0)
    def _():
        pltpu.sync_copy(zeros_hbm, acc_sh.at[sl])

    plsc.subcore_barrier()

    def i_descs(j, k):
        return (pltpu.make_async_copy(src_hbm.at[c, s, j], src_v.at[k], isem[k]),
                pltpu.make_async_copy(dst_hbm.at[c, s, j], dst_v.at[k], isem[k]))

    def g_desc(r, k):
        return pltpu.make_async_copy(
            y_hbm.at[src_v.at[k]], rows_v.at[r], gsem[r])

    def s_desc(r, k):
        return pltpu.make_async_copy(
            rows_v.at[r], acc_sh.at[dst_v.at[k]], ssem[r])

    def start_idx(j, k):
        d1, d2 = i_descs(j, k)
        d1.start()
        d2.start()

    def wait_idx(j, k):
        d1, d2 = i_descs(j, k)
        d1.wait()
        d2.wait()

    def step(k, j0, first, last):
        r = k % R
        g_desc(r, k).wait()
        s_desc(r, k).start(add=True)
        rp = (k - L) % R
        kp = (k - L) % _IRING
        if not (first and k < L):
            s_desc(rp, kp).wait()
            if (not last) or k < L:
                start_idx(j0 + k - L + _IRING, kp)
        if not (last and k >= _IRING - (R - L)):
            kn = (k + R - L) % _IRING
            wait_idx(j0 + k + R - L, kn)
            g_desc(rp, kn).start()

    for k in range(_IRING):
        start_idx(k, k)
    for m in range(R - L):
        wait_idx(m, m)
        g_desc(m % R, m).start()

    for k in range(_IRING):
        step(k, 0, True, cpt == _IRING)

    def body(i, carry):
        j0 = i * _IRING
        for k in range(_IRING):
            step(k, j0, False, False)
        return carry

    lax.fori_loop(1, cpt // _IRING - 1, body, 0)
    for k in range(_IRING):
        step(k, cpt - _IRING, False, True)
    for m in range(L):
        s_desc((_IRING - L + m) % R, _IRING - L + m).wait()
    plsc.subcore_barrier()
    pltpu.sync_copy(acc_sh.at[sl], out_hbm.at[c, sl])


def _make_agg(D, R, L, cpt, chunk, tc_tiling):
    return functools.partial(
        pl.kernel,
        out_type=jax.ShapeDtypeStruct((NC, ROWS, D), jnp.float32),
        mesh=_MESH,
        compiler_params=(None if tc_tiling
                         else pltpu.CompilerParams(use_tc_tiling_on_sc=False)),
        scratch_types=[
            pltpu.VMEM((_IRING, chunk), jnp.int32),
            pltpu.VMEM((_IRING, chunk), jnp.int32),
            pltpu.VMEM((R, chunk, D), jnp.float32),
            pltpu.VMEM_SHARED((ROWS, D), jnp.float32),
        ] + [pltpu.SemaphoreType.DMA] * (_IRING + 2 * R),
    )(functools.partial(_sc_agg, R, L, cpt))


_agg128_kernel = _make_agg(D_HID, 2, 0, CPT, CHUNK, True)
_agg64_kernel = _make_agg(D_CLS_PAD, 4, 0, CPT, CHUNK, False)


_BLK = 2048
_GRID = ROWS // _BLK
_OBLK = 400
_OGRID = N_REAL // _OBLK


def _dis(degt_block):
    return lax.rsqrt(degt_block[:, 0] + degt_block[:, 1] + 1.0)[:, None]


def _tc_mm1(x_ref, w_ref, degt_ref, y_ref):
    y_ref[...] = jnp.dot(x_ref[...], w_ref[...],
                         preferred_element_type=jnp.float32) * _dis(degt_ref[...])


def _tc_hidden(agg_ref, degt_ref, b1_ref, w2_ref, y2_ref):
    i = pl.program_id(0)
    dis = _dis(degt_ref[...])
    h = (agg_ref[0] + agg_ref[1]) * dis + b1_ref[...]
    h = jnp.maximum(h, 0.0)
    rows = i * _BLK + lax.broadcasted_iota(jnp.int32, (_BLK, 1), 0)
    mask = (rows < N_REAL).astype(jnp.float32)
    y2_ref[...] = jnp.dot(h, w2_ref[...],
                          preferred_element_type=jnp.float32) * dis * mask


def _tc_out(agg_ref, degt_ref, b2_ref, o_ref):
    dis = _dis(degt_ref[...])
    z = ((agg_ref[0] + agg_ref[1]) * dis)[:, :D_CLS] + b2_ref[...]
    m = jnp.max(z, axis=1, keepdims=True)
    lse = jnp.log(jnp.sum(jnp.exp(z - m), axis=1, keepdims=True)) + m
    o_ref[...] = z - lse


def _mm1_call(x_pad, W1, degt):
    return pl.pallas_call(
        _tc_mm1,
        grid=(_GRID,),
        in_specs=[
            pl.BlockSpec((_BLK, D_IN), lambda i: (i, 0)),
            pl.BlockSpec((D_IN, D_HID), lambda i: (0, 0)),
            pl.BlockSpec((_BLK, NC), lambda i: (i, 0)),
        ],
        out_specs=pl.BlockSpec((_BLK, D_HID), lambda i: (i, 0)),
        out_shape=jax.ShapeDtypeStruct((ROWS, D_HID), jnp.float32),
    )(x_pad, W1, degt)


def _hidden_call(agg1, degt, b1, W2p):
    return pl.pallas_call(
        _tc_hidden,
        grid=(_GRID,),
        in_specs=[
            pl.BlockSpec((NC, _BLK, D_HID), lambda i: (0, i, 0)),
            pl.BlockSpec((_BLK, NC), lambda i: (i, 0)),
            pl.BlockSpec((1, D_HID), lambda i: (0, 0)),
            pl.BlockSpec((D_HID, D_CLS_PAD), lambda i: (0, 0)),
        ],
        out_specs=pl.BlockSpec((_BLK, D_CLS_PAD), lambda i: (i, 0)),
        out_shape=jax.ShapeDtypeStruct((ROWS, D_CLS_PAD), jnp.float32),
    )(agg1, degt, b1, W2p)


def _out_call(agg2, degt, b2):
    return pl.pallas_call(
        _tc_out,
        grid=(_OGRID,),
        in_specs=[
            pl.BlockSpec((NC, _OBLK, D_CLS_PAD), lambda i: (0, i, 0)),
            pl.BlockSpec((_OBLK, NC), lambda i: (i, 0)),
            pl.BlockSpec((1, D_CLS), lambda i: (0, 0)),
        ],
        out_specs=pl.BlockSpec((_OBLK, D_CLS), lambda i: (i, 0)),
        out_shape=jax.ShapeDtypeStruct((N_REAL, D_CLS), jnp.float32),
    )(agg2, degt, b2)


def kernel(x, edge_index, W1, b1, W2, b2):
    ei = edge_index.astype(jnp.int32)
    src = ei[0]
    dst = ei[1]
    pad_n = E_PAD - src.shape[0]
    pad_iota = jnp.arange(pad_n, dtype=jnp.int32)
    pad_rows = N_REAL + pad_iota % DUMMY
    srcp = jnp.concatenate([src, pad_rows]).reshape(NC, NS, CPT, CHUNK)
    dstp = jnp.concatenate([dst, pad_rows]).reshape(NC, NS, CPT, CHUNK)

    zeros_deg = jnp.zeros((RPT,), jnp.float32)
    zeros128 = jnp.zeros((RPT, D_HID), jnp.float32)
    zeros64 = jnp.zeros((RPT, D_CLS_PAD), jnp.float32)
    x_pad = jnp.concatenate([x, jnp.zeros((DUMMY, D_IN), jnp.float32)])
    W2p = jnp.concatenate(
        [W2, jnp.zeros((D_HID, D_CLS_PAD - D_CLS), jnp.float32)], axis=1)

    degp = _degree_kernel(dstp, zeros_deg)
    degt = degp.T
    y1 = _mm1_call(x_pad, W1, degt)
    agg1 = _agg128_kernel(srcp, dstp, y1, zeros128)
    y2 = _hidden_call(agg1, degt, b1.reshape(1, D_HID), W2p)
    agg2 = _agg64_kernel(srcp, dstp, y2, zeros64)
    return _out_call(agg2, degt, b2.reshape(1, D_CLS))
